# baseline (speedup 1.0000x reference)
import functools

import jax
import jax.numpy as jnp
from jax import lax
from jax.experimental import pallas as pl
from jax.experimental.pallas import tpu as pltpu
from jax.experimental.pallas import tpu_sc as plsc

NUM_CORES = 2
NUM_SUBCORES = 16
NUM_WORKERS = NUM_CORES * NUM_SUBCORES
NBUF = 16
T_PAD = 32
D_PAD = 128


@functools.lru_cache(maxsize=None)
def _make_gather(S, T, D, dtype_name):
    dtype = jnp.dtype(dtype_name)
    rows_per_w = S // NUM_WORKERS
    n_chunks = rows_per_w
    n_outer = n_chunks // NBUF
    assert rows_per_w * NUM_WORKERS == S
    assert n_outer * NBUF == n_chunks

    mesh = plsc.VectorSubcoreMesh(core_axis_name="c", subcore_axis_name="s")

    @functools.partial(
        pl.kernel,
        mesh=mesh,
        out_type=jax.ShapeDtypeStruct((S, T_PAD, D_PAD), dtype),
        scratch_types=(
            [pltpu.VMEM((rows_per_w, T), jnp.int32)]
            + [pltpu.VMEM((T, D), dtype) for _ in range(NBUF)]
            + [pltpu.SemaphoreType.DMA for _ in range(NBUF)]
        ),
        compiler_params=pltpu.CompilerParams(use_tc_tiling_on_sc=False,
                                             needs_layout_passes=False),
    )
    def gather(table_hbm, idx_hbm, out_hbm, idx_v, *rest):
        bufs = rest[:NBUF]
        sems = rest[NBUF:]
        wid = lax.axis_index("s") * NUM_CORES + lax.axis_index("c")
        row0 = wid * rows_per_w

        pltpu.sync_copy(idx_hbm.at[pl.ds(row0, rows_per_w)], idx_v)

        def fire(j, b):
            pltpu.async_copy(table_hbm.at[idx_v.at[j]], bufs[b], sems[b])

        def drain(j, b):
            pltpu.make_async_copy(table_hbm.at[idx_v.at[j]], bufs[b],
                                  sems[b]).wait()
            pltpu.sync_copy(bufs[b],
                            out_hbm.at[row0 + j, pl.ds(0, T), pl.ds(0, D)])

        for b in range(NBUF):
            fire(b, b)

        def outer(g, carry):
            for b in range(NBUF):
                j = g * NBUF + b
                drain(j, b)
                fire(j + NBUF, b)
            return carry

        if n_outer > 1:
            lax.fori_loop(0, n_outer - 1, outer, 0)

        for j in range((n_outer - 1) * NBUF, n_chunks):
            drain(j, j % NBUF)

    return gather


def kernel(weight, indices):
    S, T = indices.shape
    D = weight.shape[1]
    outp = _make_gather(S, T, D, str(weight.dtype))(
        weight, indices.astype(jnp.int32))
    return outp[:, :T, :D]


# speedup vs baseline: 1.0027x; 1.0027x over previous
import functools

import jax
import jax.numpy as jnp
from jax import lax
from jax.experimental import pallas as pl
from jax.experimental.pallas import tpu as pltpu
from jax.experimental.pallas import tpu_sc as plsc

NUM_CORES = 2
NUM_SUBCORES = 16
NUM_WORKERS = NUM_CORES * NUM_SUBCORES
NBUF = 16
T_PAD = 32
D_PAD = 128


@functools.lru_cache(maxsize=None)
def _make_gather(S, T, D, dtype_name):
    dtype = jnp.dtype(dtype_name)
    rows_per_w = S // NUM_WORKERS
    n_chunks = rows_per_w
    n_outer = n_chunks // NBUF
    assert rows_per_w * NUM_WORKERS == S
    assert n_outer * NBUF == n_chunks
    assert T <= T_PAD and D <= D_PAD

    mesh = plsc.VectorSubcoreMesh(core_axis_name="c", subcore_axis_name="s")

    @functools.partial(
        pl.kernel,
        mesh=mesh,
        out_type=jax.ShapeDtypeStruct((S, T_PAD, D_PAD), dtype),
        scratch_types=(
            [pltpu.VMEM((rows_per_w, T), jnp.int32)]
            + [pltpu.VMEM((T, D), dtype) for _ in range(NBUF)]
            + [pltpu.SemaphoreType.DMA for _ in range(NBUF)]
        ),
        compiler_params=pltpu.CompilerParams(use_tc_tiling_on_sc=False,
                                             needs_layout_passes=False),
    )
    def gather(table_hbm, idx_hbm, out_hbm, idx_v, *rest):
        bufs = rest[:NBUF]
        sems = rest[NBUF:]
        wid = lax.axis_index("s") * NUM_CORES + lax.axis_index("c")
        row0 = wid * rows_per_w

        pltpu.sync_copy(idx_hbm.at[pl.ds(row0, rows_per_w)], idx_v)

        def fire(j, b):
            pltpu.async_copy(table_hbm.at[idx_v.at[j]], bufs[b], sems[b])

        def drain(j, b):
            pltpu.make_async_copy(table_hbm.at[idx_v.at[j]], bufs[b],
                                  sems[b]).wait()
            pltpu.sync_copy(bufs[b],
                            out_hbm.at[row0 + j, pl.ds(0, T), pl.ds(0, D)])

        for b in range(NBUF):
            fire(b, b)

        def outer(g, carry):
            for b in range(NBUF):
                j = g * NBUF + b
                drain(j, b)
                fire(j + NBUF, b)
            return carry

        if n_outer > 1:
            lax.fori_loop(0, n_outer - 1, outer, 0)

        for j in range((n_outer - 1) * NBUF, n_chunks):
            drain(j, j % NBUF)

    return gather


def kernel(weight, indices):
    S, T = indices.shape
    D = weight.shape[1]
    outp = _make_gather(S, T, D, str(weight.dtype))(
        weight, indices.astype(jnp.int32))
    return outp[:, :T, :D]
